# baseline (speedup 1.0000x reference)
import jax
import jax.numpy as jnp
from jax.experimental import pallas as pl
from jax.experimental.pallas import tpu as pltpu

_N = 10000
_D = 128
_BR = 80


def _gcn_body(a_ref, xbf_ref, xblk_ref, w_ref, b_ref, al_ref, o_ref):
    ax = jnp.dot(a_ref[...].astype(jnp.bfloat16), xbf_ref[...],
                 preferred_element_type=jnp.float32)
    axw = jnp.dot(ax.astype(jnp.bfloat16), w_ref[...],
                  preferred_element_type=jnp.float32)
    a0 = al_ref[0, 0]
    a1 = al_ref[0, 1]
    m = jnp.maximum(a0, a1)
    e0 = jnp.exp(a0 - m)
    e1 = jnp.exp(a1 - m)
    g0 = e0 / (e0 + e1)
    g1 = e1 / (e0 + e1)
    y = g0 * axw + g1 * xblk_ref[...] + b_ref[...]
    o_ref[...] = jnp.where(y > 0.0, y, jnp.exp(jnp.minimum(y, 0.0)) - 1.0)


def kernel(inputs, adj, weight, bias, alpha):
    x_bf = inputs.astype(jnp.bfloat16)
    w_bf = weight.astype(jnp.bfloat16)
    bias2 = bias.reshape(1, _D)
    al2 = alpha.reshape(1, 2)
    grid = (_N // _BR,)
    return pl.pallas_call(
        _gcn_body,
        grid=grid,
        in_specs=[
            pl.BlockSpec((_BR, _N), lambda i: (i, 0)),
            pl.BlockSpec((_N, _D), lambda i: (0, 0)),
            pl.BlockSpec((_BR, _D), lambda i: (i, 0)),
            pl.BlockSpec((_D, _D), lambda i: (0, 0)),
            pl.BlockSpec((1, _D), lambda i: (0, 0)),
            pl.BlockSpec((1, 2), lambda i: (0, 0)),
        ],
        out_specs=pl.BlockSpec((_BR, _D), lambda i: (i, 0)),
        out_shape=jax.ShapeDtypeStruct((_N, _D), jnp.float32),
        compiler_params=pltpu.CompilerParams(
            dimension_semantics=("parallel",),
        ),
    )(adj, x_bf, inputs, w_bf, bias2, al2)


# speedup vs baseline: 1.4276x; 1.4276x over previous
import jax
import jax.numpy as jnp
from jax.experimental import pallas as pl
from jax.experimental.pallas import tpu as pltpu

_N = 10000
_D = 128
_BH = 200
_BR = 2 * _BH


def _gcn_body(a0_ref, a1_ref, xbf_ref, xblk_ref, w_ref, b_ref, al_ref, o_ref):
    ax0 = jnp.dot(a0_ref[...].astype(jnp.bfloat16), xbf_ref[...],
                  preferred_element_type=jnp.float32)
    ax1 = jnp.dot(a1_ref[...].astype(jnp.bfloat16), xbf_ref[...],
                  preferred_element_type=jnp.float32)
    ax = jnp.concatenate([ax0, ax1], axis=0)
    axw = jnp.dot(ax.astype(jnp.bfloat16), w_ref[...],
                  preferred_element_type=jnp.float32)
    a0 = al_ref[0, 0]
    a1 = al_ref[0, 1]
    m = jnp.maximum(a0, a1)
    e0 = jnp.exp(a0 - m)
    e1 = jnp.exp(a1 - m)
    g0 = e0 / (e0 + e1)
    g1 = e1 / (e0 + e1)
    y = g0 * axw + g1 * xblk_ref[...] + b_ref[...]
    o_ref[...] = jnp.where(y > 0.0, y, jnp.exp(jnp.minimum(y, 0.0)) - 1.0)


def kernel(inputs, adj, weight, bias, alpha):
    x_bf = inputs.astype(jnp.bfloat16)
    w_bf = weight.astype(jnp.bfloat16)
    bias2 = bias.reshape(1, _D)
    al2 = alpha.reshape(1, 2)
    grid = (_N // _BR,)
    return pl.pallas_call(
        _gcn_body,
        grid=grid,
        in_specs=[
            pl.BlockSpec((_BH, _N), lambda i: (2 * i, 0)),
            pl.BlockSpec((_BH, _N), lambda i: (2 * i + 1, 0)),
            pl.BlockSpec((_N, _D), lambda i: (0, 0)),
            pl.BlockSpec((_BR, _D), lambda i: (i, 0)),
            pl.BlockSpec((_D, _D), lambda i: (0, 0)),
            pl.BlockSpec((1, _D), lambda i: (0, 0)),
            pl.BlockSpec((1, 2), lambda i: (0, 0)),
        ],
        out_specs=pl.BlockSpec((_BR, _D), lambda i: (i, 0)),
        out_shape=jax.ShapeDtypeStruct((_N, _D), jnp.float32),
        compiler_params=pltpu.CompilerParams(
            dimension_semantics=("parallel",),
        ),
    )(adj, adj, x_bf, inputs, w_bf, bias2, al2)
